# baseline (speedup 1.0000x reference)
import functools
import math

import jax
import jax.numpy as jnp
from jax import lax
from jax.experimental import pallas as pl
from jax.experimental.pallas import tpu as pltpu
from jax.experimental.pallas import tpu_sc as plsc

_DIM = 64
_SCALE = math.sqrt(_DIM)

_NC = 2
_NS = 16
_NW = _NC * _NS
_IW = 128
_CH = 1024


@functools.cache
def _build(B):
  assert B % (_NW * _CH) == 0
  b_per_w = B // _NW
  n_chunks = b_per_w // _CH
  n_g = _CH // _IW

  mesh = plsc.VectorSubcoreMesh(core_axis_name="c", subcore_axis_name="s")

  def body(x_hbm, table_hbm, out_hbm, idx_v, rows_v, gsem):
    wid = lax.axis_index("s") * _NC + lax.axis_index("c")
    base = wid * b_per_w

    def chunk(i, carry):
      off = pl.multiple_of(base + i * _CH, _CH)
      pltpu.sync_copy(x_hbm.at[pl.ds(pl.multiple_of(off // _IW, 8), n_g)], idx_v)
      copies = [
          pltpu.async_copy(
              table_hbm.at[idx_v.at[j]],
              rows_v.at[pl.ds(j * _IW, _IW)],
              gsem,
          )
          for j in range(n_g)
      ]
      for cp in copies:
        cp.wait()

      def srow(r, c):
        for rr in range(2):
          for u in range(_DIM // 16):
            sl = pl.ds(u * 16, 16)
            rows_v[r * 2 + rr, sl] = rows_v[r * 2 + rr, sl] * _SCALE
        return c

      lax.fori_loop(0, _CH // 2, srow, 0)

      pltpu.sync_copy(rows_v, out_hbm.at[pl.ds(off, _CH)])
      return carry

    lax.fori_loop(0, n_chunks, chunk, 0)

  return pl.kernel(
      body,
      out_type=jax.ShapeDtypeStruct((B, _DIM), jnp.float32),
      mesh=mesh,
      compiler_params=pltpu.CompilerParams(use_tc_tiling_on_sc=False),
      scratch_types=[
          pltpu.VMEM((n_g, _IW), jnp.int32),
          pltpu.VMEM((_CH, _DIM), jnp.float32),
          pltpu.SemaphoreType.DMA,
      ],
  )


def kernel(x, table):
  B = x.size
  x2 = x.reshape(B // _IW, _IW).astype(jnp.int32)
  out = _build(B)(x2, table.astype(jnp.float32))
  return out.reshape(*x.shape, _DIM)


# speedup vs baseline: 1.0635x; 1.0635x over previous
import functools
import math

import jax
import jax.numpy as jnp
from jax import lax
from jax.experimental import pallas as pl
from jax.experimental.pallas import tpu as pltpu
from jax.experimental.pallas import tpu_sc as plsc

_DIM = 64
_SCALE = math.sqrt(_DIM)

_NC = 2
_NS = 16
_NW = _NC * _NS
_IW = 128
_CH = 256
_NB = 4
_NG = _CH // _IW


@functools.cache
def _build(B):
  assert B % (_NW * _CH) == 0
  b_per_w = B // _NW
  n_chunks = b_per_w // _CH
  n_idx_rows = b_per_w // _IW
  assert n_chunks % _NB == 0 and n_chunks // _NB >= 2

  mesh = plsc.VectorSubcoreMesh(core_axis_name="c", subcore_axis_name="s")

  def body(x_hbm, table_hbm, out_hbm, idx_v, bufs, gsems, wsems):
    wid = lax.axis_index("s") * _NC + lax.axis_index("c")
    base = pl.multiple_of(wid * b_per_w, b_per_w)

    pltpu.sync_copy(
        x_hbm.at[pl.ds(pl.multiple_of(base // _IW, 8), n_idx_rows)], idx_v)

    def gather(g, nb):
      for j in range(_NG):
        pltpu.make_async_copy(
            table_hbm.at[idx_v.at[g * _NG + j]],
            bufs[nb].at[pl.ds(j * _IW, _IW)],
            gsems[nb],
        ).start()

    def gather_wait(g, nb):
      for j in range(_NG):
        pltpu.make_async_copy(
            table_hbm.at[idx_v.at[g * _NG + j]],
            bufs[nb].at[pl.ds(j * _IW, _IW)],
            gsems[nb],
        ).wait()

    def out_slot(g):
      return out_hbm.at[pl.ds(pl.multiple_of(base + g * _CH, 8), _CH)]

    def wout(g, nb):
      pltpu.make_async_copy(bufs[nb], out_slot(g), wsems[nb]).start()

    def wout_wait(g, nb):
      pltpu.make_async_copy(bufs[nb], out_slot(g), wsems[nb]).wait()

    def scale(nb):
      def srow(r, c):
        for rr in range(2):
          for u in range(_DIM // 16):
            sl = pl.ds(u * 16, 16)
            bufs[nb][r * 2 + rr, sl] = bufs[nb][r * 2 + rr, sl] * _SCALE
        return c

      lax.fori_loop(0, _CH // 2, srow, 0)

    def chunk(g, nb, fire_ahead, wait_prev=True):
      gather_wait(g, nb)
      scale(nb)
      wout(g, nb)
      if fire_ahead:
        nb2 = (nb + _NB - 1) % _NB
        if wait_prev:
          wout_wait(g - 1, nb2)
        gather(g + _NB - 1, nb2)

    for nb in range(_NB - 1):
      gather(nb, nb)

    chunk(0, 0, True, wait_prev=False)
    for nb in range(1, _NB):
      chunk(nb, nb, True)

    def outer(i, carry):
      g0 = i * _NB
      for nb in range(_NB):
        chunk(g0 + nb, nb, True)
      return carry

    lax.fori_loop(1, n_chunks // _NB - 1, outer, 0)

    g0 = n_chunks - _NB
    chunk(g0, 0, True)
    for nb in range(1, _NB):
      chunk(g0 + nb, nb, False)
    for nb in range(_NB):
      wout_wait(g0 + nb, nb)

  return pl.kernel(
      body,
      out_type=jax.ShapeDtypeStruct((B, _DIM), jnp.float32),
      mesh=mesh,
      compiler_params=pltpu.CompilerParams(use_tc_tiling_on_sc=False),
      scratch_types=[
          pltpu.VMEM((b_per_w // _IW, _IW), jnp.int32),
          [pltpu.VMEM((_CH, _DIM), jnp.float32) for _ in range(_NB)],
          [pltpu.SemaphoreType.DMA for _ in range(_NB)],
          [pltpu.SemaphoreType.DMA for _ in range(_NB)],
      ],
  )


def kernel(x, table):
  B = x.size
  x2 = x.reshape(B // _IW, _IW).astype(jnp.int32)
  out = _build(B)(x2, table.astype(jnp.float32))
  return out.reshape(*x.shape, _DIM)
